# baseline (speedup 1.0000x reference)
import jax
import jax.numpy as jnp
from jax import lax
from jax.experimental import pallas as pl
from jax.experimental.pallas import tpu as pltpu

B_, L, DM = 2, 2048, 1024
DI, N, K, R = 2048, 16, 4, 64
EPS = 1e-5
C = 256


def _ssm_kernel(x_ref, nw_ref, win_ref, cwt_ref, cb_ref, wx_ref, wdt_ref,
                dtb_ref, alogt_ref, dD_ref, wout_ref, o_ref,
                h_ref, carry_ref, dts_ref, gs_ref, bts_ref, cts_ref, ys_ref,
                at_ref, us_ref, zs_ref):
    j = pl.program_id(1)

    @pl.when(j == 0)
    def _():
        h_ref[...] = jnp.zeros_like(h_ref)
        carry_ref[...] = jnp.zeros_like(carry_ref)

    xb = x_ref[0]

    var = jnp.mean(xb * xb, axis=-1, keepdims=True)
    hn = xb * lax.rsqrt(var + EPS) * nw_ref[...]

    xz = jnp.dot(hn, win_ref[...], preferred_element_type=jnp.float32)
    u_pre = xz[:, :DI]
    z = xz[:, DI:]

    full = jnp.concatenate([carry_ref[...], u_pre], axis=0)
    carry_ref[...] = u_pre[C - (K - 1):, :]
    uc = cb_ref[...]
    for k in range(K):
        uc = uc + full[k:k + C, :] * cwt_ref[k:k + 1, :]
    u = uc * jax.nn.sigmoid(uc)
    us_ref[...] = u
    zs_ref[...] = z

    xdbl = jnp.dot(u, wx_ref[...], preferred_element_type=jnp.float32)
    dt = jax.nn.softplus(
        jnp.dot(xdbl[:, :R], wdt_ref[...], preferred_element_type=jnp.float32)
        + dtb_ref[...])

    dts_ref[...] = dt
    gs_ref[...] = dt * u
    bts_ref[...] = xdbl[:, R:R + N]
    cts_ref[...] = xdbl[:, R + N:R + 2 * N]

    at_ref[...] = -jnp.exp(alogt_ref[...])

    G = 4

    def step(i, h):
        base = i * G
        bcm = jnp.transpose(bts_ref[pl.ds(base, G), :])
        ccm = jnp.transpose(cts_ref[pl.ds(base, G), :])
        for g in range(G):
            dtv = dts_ref[pl.ds(base + g, 1), :]
            gv = gs_ref[pl.ds(base + g, 1), :]
            h = jnp.exp(dtv * at_ref[...]) * h + bcm[:, g:g + 1] * gv
            ys_ref[pl.ds(base + g, 1), :] = jnp.sum(
                h * ccm[:, g:g + 1], axis=0, keepdims=True)
        return h

    h = lax.fori_loop(0, C // G, step, h_ref[...])
    h_ref[...] = h

    y = ys_ref[...] + us_ref[...] * dD_ref[...]
    z2 = zs_ref[...]
    y = y * (z2 * jax.nn.sigmoid(z2))

    o_ref[0] = x_ref[0] + jnp.dot(y, wout_ref[...],
                                  preferred_element_type=jnp.float32)


def _ssm_fused(x, norm_w, in_proj_w, conv_w, conv_b, x_proj_w, dt_proj_w,
               dt_proj_b, A_log, D, out_proj_w, interpret=False):
    nw = norm_w.reshape(1, DM)
    cwt = jnp.transpose(conv_w)
    cb = conv_b.reshape(1, DI)
    dtb = dt_proj_b.reshape(1, DI)
    alogt = jnp.transpose(A_log)
    dD = D.reshape(1, DI)

    const = lambda b, j: (0, 0)
    return pl.pallas_call(
        _ssm_kernel,
        out_shape=jax.ShapeDtypeStruct((B_, L, DM), jnp.float32),
        grid=(B_, L // C),
        in_specs=[
            pl.BlockSpec((1, C, DM), lambda b, j: (b, j, 0)),
            pl.BlockSpec((1, DM), const),
            pl.BlockSpec((DM, 2 * DI), const),
            pl.BlockSpec((K, DI), const),
            pl.BlockSpec((1, DI), const),
            pl.BlockSpec((DI, R + 2 * N), const),
            pl.BlockSpec((R, DI), const),
            pl.BlockSpec((1, DI), const),
            pl.BlockSpec((N, DI), const),
            pl.BlockSpec((1, DI), const),
            pl.BlockSpec((DI, DM), const),
        ],
        out_specs=pl.BlockSpec((1, C, DM), lambda b, j: (b, j, 0)),
        scratch_shapes=[
            pltpu.VMEM((N, DI), jnp.float32),
            pltpu.VMEM((K - 1, DI), jnp.float32),
            pltpu.VMEM((C, DI), jnp.float32),
            pltpu.VMEM((C, DI), jnp.float32),
            pltpu.VMEM((C, N), jnp.float32),
            pltpu.VMEM((C, N), jnp.float32),
            pltpu.VMEM((C, DI), jnp.float32),
            pltpu.VMEM((N, DI), jnp.float32),
            pltpu.VMEM((C, DI), jnp.float32),
            pltpu.VMEM((C, DI), jnp.float32),
        ],
        compiler_params=pltpu.CompilerParams(
            dimension_semantics=("arbitrary", "arbitrary"),
            vmem_limit_bytes=56 * 1024 * 1024,
            flags={"XLA_TPU_STORE_TO_LOAD_FORWARDING_WINDOW": 24576},
        ),
        name="ssm_layer_fused",
        interpret=interpret,
    )(x, nw, in_proj_w, cwt, cb, x_proj_w, dt_proj_w, dtb, alogt, dD,
      out_proj_w)


def kernel(x, hormone_vectors, norm_w, in_proj_w, conv_w, conv_b, x_proj_w,
           dt_proj_w, dt_proj_b, A_log, D, out_proj_w):
    del hormone_vectors
    return _ssm_fused(x, norm_w, in_proj_w, conv_w, conv_b, x_proj_w,
                      dt_proj_w, dt_proj_b, A_log, D, out_proj_w)


# speedup vs baseline: 1.0024x; 1.0024x over previous
import jax
import jax.numpy as jnp
from jax import lax
from jax.experimental import pallas as pl
from jax.experimental.pallas import tpu as pltpu

B_, L, DM = 2, 2048, 1024
DI, N, K, R = 2048, 16, 4, 64
EPS = 1e-5
C = 256


def _ssm_kernel(x_ref, nw_ref, win_ref, cwt_ref, cb_ref, wx_ref, wdt_ref,
                dtb_ref, alogt_ref, dD_ref, wout_ref, o_ref,
                h_ref, carry_ref, dts_ref, gs_ref, bts_ref, cts_ref, ys_ref,
                at_ref, us_ref, zs_ref):
    j = pl.program_id(1)

    @pl.when(j == 0)
    def _():
        h_ref[...] = jnp.zeros_like(h_ref)
        carry_ref[...] = jnp.zeros_like(carry_ref)

    xb = x_ref[0]

    var = jnp.mean(xb * xb, axis=-1, keepdims=True)
    hn = xb * lax.rsqrt(var + EPS) * nw_ref[...]

    xz = jnp.dot(hn, win_ref[...], preferred_element_type=jnp.float32)
    u_pre = xz[:, :DI]
    z = xz[:, DI:]

    full = jnp.concatenate([carry_ref[...], u_pre], axis=0)
    carry_ref[...] = u_pre[C - (K - 1):, :]
    uc = cb_ref[...]
    for k in range(K):
        uc = uc + full[k:k + C, :] * cwt_ref[k:k + 1, :]
    u = uc * jax.nn.sigmoid(uc)
    us_ref[...] = u
    zs_ref[...] = z

    xdbl = jnp.dot(u, wx_ref[...], preferred_element_type=jnp.float32)
    dt = jax.nn.softplus(
        jnp.dot(xdbl[:, :R], wdt_ref[...], preferred_element_type=jnp.float32)
        + dtb_ref[...])

    dts_ref[...] = dt
    gs_ref[...] = dt * u
    bts_ref[...] = xdbl[:, R:R + N]
    cts_ref[...] = xdbl[:, R + N:R + 2 * N]

    at_ref[...] = -jnp.exp(alogt_ref[...])

    G = 4

    def step(i, h):
        base = i * G
        bcm = jnp.transpose(bts_ref[pl.ds(base, G), :])
        ccm = jnp.transpose(cts_ref[pl.ds(base, G), :])
        for g in range(G):
            dtv = dts_ref[pl.ds(base + g, 1), :]
            gv = gs_ref[pl.ds(base + g, 1), :]
            h = jnp.exp(dtv * at_ref[...]) * h + bcm[:, g:g + 1] * gv
            ys_ref[pl.ds(base + g, 1), :] = jnp.sum(
                h * ccm[:, g:g + 1], axis=0, keepdims=True)
        return h

    h = lax.fori_loop(0, C // G, step, h_ref[...])
    h_ref[...] = h

    y = ys_ref[...] + us_ref[...] * dD_ref[...]
    z2 = zs_ref[...]
    y = y * (z2 * jax.nn.sigmoid(z2))

    o_ref[0] = x_ref[0] + jnp.dot(y, wout_ref[...],
                                  preferred_element_type=jnp.float32)


def _ssm_fused(x, norm_w, in_proj_w, conv_w, conv_b, x_proj_w, dt_proj_w,
               dt_proj_b, A_log, D, out_proj_w, interpret=False):
    nw = norm_w.reshape(1, DM)
    cwt = jnp.transpose(conv_w)
    cb = conv_b.reshape(1, DI)
    dtb = dt_proj_b.reshape(1, DI)
    alogt = jnp.transpose(A_log)
    dD = D.reshape(1, DI)

    const = lambda b, j: (0, 0)
    return pl.pallas_call(
        _ssm_kernel,
        out_shape=jax.ShapeDtypeStruct((B_, L, DM), jnp.float32),
        grid=(B_, L // C),
        in_specs=[
            pl.BlockSpec((1, C, DM), lambda b, j: (b, j, 0)),
            pl.BlockSpec((1, DM), const),
            pl.BlockSpec((DM, 2 * DI), const),
            pl.BlockSpec((K, DI), const),
            pl.BlockSpec((1, DI), const),
            pl.BlockSpec((DI, R + 2 * N), const),
            pl.BlockSpec((R, DI), const),
            pl.BlockSpec((1, DI), const),
            pl.BlockSpec((N, DI), const),
            pl.BlockSpec((1, DI), const),
            pl.BlockSpec((DI, DM), const),
        ],
        out_specs=pl.BlockSpec((1, C, DM), lambda b, j: (b, j, 0)),
        scratch_shapes=[
            pltpu.VMEM((N, DI), jnp.float32),
            pltpu.VMEM((K - 1, DI), jnp.float32),
            pltpu.VMEM((C, DI), jnp.float32),
            pltpu.VMEM((C, DI), jnp.float32),
            pltpu.VMEM((C, N), jnp.float32),
            pltpu.VMEM((C, N), jnp.float32),
            pltpu.VMEM((C, DI), jnp.float32),
            pltpu.VMEM((N, DI), jnp.float32),
            pltpu.VMEM((C, DI), jnp.float32),
            pltpu.VMEM((C, DI), jnp.float32),
        ],
        compiler_params=pltpu.CompilerParams(
            dimension_semantics=("arbitrary", "arbitrary"),
            vmem_limit_bytes=56 * 1024 * 1024,
            flags={"XLA_TPU_STORE_TO_LOAD_FORWARDING_WINDOW": 12288},
        ),
        name="ssm_layer_fused",
        interpret=interpret,
    )(x, nw, in_proj_w, cwt, cb, x_proj_w, dt_proj_w, dtb, alogt, dD,
      out_proj_w)


def kernel(x, hormone_vectors, norm_w, in_proj_w, conv_w, conv_b, x_proj_w,
           dt_proj_w, dt_proj_b, A_log, D, out_proj_w):
    del hormone_vectors
    return _ssm_fused(x, norm_w, in_proj_w, conv_w, conv_b, x_proj_w,
                      dt_proj_w, dt_proj_b, A_log, D, out_proj_w)
